# baseline (speedup 1.0000x reference)
import functools

import jax
import jax.numpy as jnp
from jax import lax
from jax.experimental import pallas as pl
from jax.experimental.pallas import tpu as pltpu
from jax.experimental.pallas import tpu_sc as plsc

EMBED_DIM = 64
BATCH = 16384

_NUM_CORES = 2
_NUM_SUBCORES = 16
_NUM_WORKERS = _NUM_CORES * _NUM_SUBCORES
_B_PER_W = BATCH // _NUM_WORKERS
_GROUP = 16
_N_GROUPS = _B_PER_W // _GROUP

_mesh = plsc.VectorSubcoreMesh(core_axis_name="c", subcore_axis_name="s")


@functools.partial(
    pl.kernel,
    out_type=(
        jax.ShapeDtypeStruct((BATCH, EMBED_DIM), jnp.float32),
        jax.ShapeDtypeStruct((BATCH, EMBED_DIM), jnp.float32),
    ),
    mesh=_mesh,
    scratch_types=[
        pltpu.VMEM((_B_PER_W,), jnp.int32),
        pltpu.VMEM((_B_PER_W,), jnp.int32),
        pltpu.VMEM((_GROUP, EMBED_DIM), jnp.float32),
        pltpu.VMEM((_GROUP, EMBED_DIM), jnp.float32),
        pltpu.SemaphoreType.DMA,
        pltpu.SemaphoreType.DMA,
    ],
)
def _lookup_kernel(ent_hbm, rel_hbm, eids_hbm, rids_hbm, out_e, out_r,
                   idx_e, idx_r, rows_e, rows_r, sem_e, sem_r):
    wid = lax.axis_index("s") * _NUM_CORES + lax.axis_index("c")
    base = wid * _B_PER_W

    pltpu.sync_copy(eids_hbm.at[pl.ds(base, _B_PER_W)], idx_e)
    pltpu.sync_copy(rids_hbm.at[pl.ds(base, _B_PER_W)], idx_r)

    def do_group(g, _):
        evals = idx_e[pl.ds(g * _GROUP, _GROUP)]
        rvals = idx_r[pl.ds(g * _GROUP, _GROUP)]
        copies = []
        for j in range(_GROUP):
            copies.append(pltpu.async_copy(
                ent_hbm.at[evals[j]], rows_e.at[j], sem_e))
            copies.append(pltpu.async_copy(
                rel_hbm.at[rvals[j]], rows_r.at[j], sem_r))
        for cp in copies:
            cp.wait()
        pltpu.sync_copy(rows_e, out_e.at[pl.ds(base + g * _GROUP, _GROUP)])
        pltpu.sync_copy(rows_r, out_r.at[pl.ds(base + g * _GROUP, _GROUP)])
        return 0

    lax.fori_loop(0, _N_GROUPS, do_group, 0)


def kernel(entity_embedding, relation_embedding, entity_ids, relation_ids):
    return _lookup_kernel(entity_embedding, relation_embedding,
                          entity_ids.astype(jnp.int32),
                          relation_ids.astype(jnp.int32))


# speedup vs baseline: 1.0324x; 1.0324x over previous
import functools

import jax
import jax.numpy as jnp
from jax import lax
from jax.experimental import pallas as pl
from jax.experimental.pallas import tpu as pltpu
from jax.experimental.pallas import tpu_sc as plsc

EMBED_DIM = 64
BATCH = 16384

_NUM_CORES = 2
_NUM_SUBCORES = 16
_NUM_WORKERS = _NUM_CORES * _NUM_SUBCORES
_B_PER_W = BATCH // _NUM_WORKERS
_GROUP = 32
_N_GROUPS = _B_PER_W // _GROUP

_mesh = plsc.VectorSubcoreMesh(core_axis_name="c", subcore_axis_name="s")


@functools.partial(
    pl.kernel,
    out_type=(
        jax.ShapeDtypeStruct((BATCH, EMBED_DIM), jnp.float32),
        jax.ShapeDtypeStruct((BATCH, EMBED_DIM), jnp.float32),
    ),
    mesh=_mesh,
    scratch_types=[
        pltpu.VMEM((_B_PER_W,), jnp.int32),
        pltpu.VMEM((_B_PER_W,), jnp.int32),
        pltpu.VMEM((_GROUP, EMBED_DIM), jnp.float32),
        pltpu.VMEM((_GROUP, EMBED_DIM), jnp.float32),
        pltpu.SemaphoreType.DMA,
        pltpu.SemaphoreType.DMA,
    ],
)
def _lookup_kernel(ent_hbm, rel_hbm, eids_hbm, rids_hbm, out_e, out_r,
                   idx_e, idx_r, rows_e, rows_r, sem_e, sem_r):
    wid = lax.axis_index("s") * _NUM_CORES + lax.axis_index("c")
    base = wid * _B_PER_W

    pltpu.sync_copy(eids_hbm.at[pl.ds(base, _B_PER_W)], idx_e)
    pltpu.sync_copy(rids_hbm.at[pl.ds(base, _B_PER_W)], idx_r)

    def do_group(g, _):
        copies = []
        for h in range(_GROUP // 16):
            evals = idx_e[pl.ds(g * _GROUP + h * 16, 16)]
            rvals = idx_r[pl.ds(g * _GROUP + h * 16, 16)]
            for j in range(16):
                copies.append(pltpu.async_copy(
                    ent_hbm.at[evals[j]], rows_e.at[h * 16 + j], sem_e))
                copies.append(pltpu.async_copy(
                    rel_hbm.at[rvals[j]], rows_r.at[h * 16 + j], sem_r))
        for cp in copies:
            cp.wait()
        pltpu.sync_copy(rows_e, out_e.at[pl.ds(base + g * _GROUP, _GROUP)])
        pltpu.sync_copy(rows_r, out_r.at[pl.ds(base + g * _GROUP, _GROUP)])
        return 0

    lax.fori_loop(0, _N_GROUPS, do_group, 0)


def kernel(entity_embedding, relation_embedding, entity_ids, relation_ids):
    return _lookup_kernel(entity_embedding, relation_embedding,
                          entity_ids.astype(jnp.int32),
                          relation_ids.astype(jnp.int32))


# speedup vs baseline: 1.0474x; 1.0146x over previous
import functools

import jax
import jax.numpy as jnp
from jax import lax
from jax.experimental import pallas as pl
from jax.experimental.pallas import tpu as pltpu
from jax.experimental.pallas import tpu_sc as plsc

EMBED_DIM = 64
BATCH = 16384

_NUM_CORES = 2
_NUM_SUBCORES = 16
_NUM_WORKERS = _NUM_CORES * _NUM_SUBCORES
_B_PER_W = BATCH // _NUM_WORKERS
_GROUP = 64
_N_GROUPS = _B_PER_W // _GROUP

_mesh = plsc.VectorSubcoreMesh(core_axis_name="c", subcore_axis_name="s")


@functools.partial(
    pl.kernel,
    out_type=(
        jax.ShapeDtypeStruct((BATCH, EMBED_DIM), jnp.float32),
        jax.ShapeDtypeStruct((BATCH, EMBED_DIM), jnp.float32),
    ),
    mesh=_mesh,
    scratch_types=[
        pltpu.VMEM((_B_PER_W,), jnp.int32),
        pltpu.VMEM((_B_PER_W,), jnp.int32),
        pltpu.VMEM((_GROUP, EMBED_DIM), jnp.float32),
        pltpu.VMEM((_GROUP, EMBED_DIM), jnp.float32),
        pltpu.SemaphoreType.DMA,
        pltpu.SemaphoreType.DMA,
    ],
)
def _lookup_kernel(ent_hbm, rel_hbm, eids_hbm, rids_hbm, out_e, out_r,
                   idx_e, idx_r, rows_e, rows_r, sem_e, sem_r):
    wid = lax.axis_index("s") * _NUM_CORES + lax.axis_index("c")
    base = wid * _B_PER_W

    pltpu.sync_copy(eids_hbm.at[pl.ds(base, _B_PER_W)], idx_e)
    pltpu.sync_copy(rids_hbm.at[pl.ds(base, _B_PER_W)], idx_r)

    def do_group(g, _):
        copies = []
        for h in range(_GROUP // 16):
            evals = idx_e[pl.ds(g * _GROUP + h * 16, 16)]
            rvals = idx_r[pl.ds(g * _GROUP + h * 16, 16)]
            for j in range(16):
                copies.append(pltpu.async_copy(
                    ent_hbm.at[evals[j]], rows_e.at[h * 16 + j], sem_e))
                copies.append(pltpu.async_copy(
                    rel_hbm.at[rvals[j]], rows_r.at[h * 16 + j], sem_r))
        for cp in copies:
            cp.wait()
        pltpu.sync_copy(rows_e, out_e.at[pl.ds(base + g * _GROUP, _GROUP)])
        pltpu.sync_copy(rows_r, out_r.at[pl.ds(base + g * _GROUP, _GROUP)])
        return 0

    lax.fori_loop(0, _N_GROUPS, do_group, 0)


def kernel(entity_embedding, relation_embedding, entity_ids, relation_ids):
    return _lookup_kernel(entity_embedding, relation_embedding,
                          entity_ids.astype(jnp.int32),
                          relation_ids.astype(jnp.int32))
